# baseline (speedup 1.0000x reference)
import functools

import jax
import jax.numpy as jnp
from jax.experimental import pallas as pl
from jax.experimental.pallas import tpu as pltpu

N_NUM_F = 128
N_CAT_F = 26
CARD_F = 16
D_FLAT_F = N_NUM_F + N_CAT_F * CARD_F
E_F = 16
D_BLK = 256
TB = 256


def _fused_body(xn_ref, xc_ref, Wg_ref, bg_ref, W1_ref, b1_ref, W2_ref,
                b2_ref, Wout_ref, bout_ref, out_ref, yacc, probs_s):
    e = pl.program_id(1)

    xn = xn_ref[...]
    xc = xc_ref[...]
    iota16 = jax.lax.broadcasted_iota(jnp.int32, (TB, CARD_F), 1)
    pieces = [xn]
    for f in range(N_CAT_F):
        pieces.append((xc[:, f:f + 1] == iota16).astype(jnp.float32))
    x = jnp.concatenate(pieces, axis=1)

    @pl.when(e == 0)
    def _():
        logits = jnp.dot(x, Wg_ref[...], preferred_element_type=jnp.float32)
        logits = logits + bg_ref[...]
        m = jnp.max(logits, axis=1, keepdims=True)
        p = jnp.exp(logits - m)
        probs_s[...] = p / jnp.sum(p, axis=1, keepdims=True)

    p = probs_s[...]
    lane = jax.lax.broadcasted_iota(jnp.int32, (TB, E_F), 1)
    m1 = jnp.max(p, axis=1, keepdims=True)
    a1 = jnp.min(jnp.where(p == m1, lane, E_F), axis=1, keepdims=True)
    p2 = jnp.where(lane == a1, -1.0, p)
    m2 = jnp.max(p2, axis=1, keepdims=True)
    a2 = jnp.min(jnp.where(p2 == m2, lane, E_F), axis=1, keepdims=True)
    gate = (jnp.where(a1 == e, m1, 0.0) + jnp.where(a2 == e, m2, 0.0)) / (m1 + m2)

    h = jnp.dot(x, W1_ref[0], preferred_element_type=jnp.float32) + b1_ref[0]
    h = jnp.maximum(h, 0.0)
    h = jnp.dot(h, W2_ref[0], preferred_element_type=jnp.float32) + b2_ref[0]
    h = jnp.maximum(h, 0.0)
    contrib = gate * h

    @pl.when(e == 0)
    def _():
        yacc[...] = contrib

    @pl.when(e > 0)
    def _():
        yacc[...] += contrib

    @pl.when(e == E_F - 1)
    def _():
        out_ref[...] = (
            jnp.dot(yacc[...], Wout_ref[...], preferred_element_type=jnp.float32)
            + bout_ref[...])


@functools.partial(jax.jit, static_argnames=("interpret",))
def _moe_fused(x_num, x_cat, Wg, bg, W1, b1, W2, b2, Wout, bout,
               interpret=False):
    B = x_num.shape[0]
    T = B // TB
    grid = (T, E_F)
    out = pl.pallas_call(
        _fused_body,
        grid=grid,
        in_specs=[
            pl.BlockSpec((TB, N_NUM_F), lambda t, e: (t, 0)),
            pl.BlockSpec((TB, N_CAT_F), lambda t, e: (t, 0)),
            pl.BlockSpec((D_FLAT_F, E_F), lambda t, e: (0, 0)),
            pl.BlockSpec((1, E_F), lambda t, e: (0, 0)),
            pl.BlockSpec((1, D_FLAT_F, D_BLK), lambda t, e: (e, 0, 0)),
            pl.BlockSpec((1, 1, D_BLK), lambda t, e: (e, 0, 0)),
            pl.BlockSpec((1, D_BLK, D_BLK), lambda t, e: (e, 0, 0)),
            pl.BlockSpec((1, 1, D_BLK), lambda t, e: (e, 0, 0)),
            pl.BlockSpec((D_BLK, 1), lambda t, e: (0, 0)),
            pl.BlockSpec((1, 1), lambda t, e: (0, 0)),
        ],
        out_specs=pl.BlockSpec((TB, 1), lambda t, e: (t, 0)),
        out_shape=jax.ShapeDtypeStruct((B, 1), jnp.float32),
        scratch_shapes=[
            pltpu.VMEM((TB, D_BLK), jnp.float32),
            pltpu.VMEM((TB, E_F), jnp.float32),
        ],
        compiler_params=pltpu.CompilerParams(
            dimension_semantics=("parallel", "arbitrary")),
        interpret=interpret,
    )(x_num, x_cat, Wg, bg.reshape(1, E_F), W1, b1.reshape(E_F, 1, D_BLK),
      W2, b2.reshape(E_F, 1, D_BLK), Wout, bout.reshape(1, 1))
    return out


def kernel(x_num, x_cat, Wg, bg, W1, b1, W2, b2, Wout, bout):
    out = _moe_fused(x_num, x_cat, Wg, bg, W1, b1, W2, b2, Wout, bout)
    return out[:, None]


# speedup vs baseline: 1.8636x; 1.8636x over previous
import functools

import jax
import jax.numpy as jnp
from jax.experimental import pallas as pl
from jax.experimental.pallas import tpu as pltpu

N_NUM_F = 128
N_CAT_F = 26
CARD_F = 16
D_FLAT_F = N_NUM_F + N_CAT_F * CARD_F
E_F = 16
D_BLK = 256
TB = 256


def _fused_body(x_ref, Wg_ref, bg_ref, W1_ref, b1_ref, W2_ref,
                b2_ref, Wout_ref, bout_ref, out_ref, yacc, probs_s):
    e = pl.program_id(1)
    x = x_ref[...]

    @pl.when(e == 0)
    def _():
        logits = jnp.dot(x, Wg_ref[...], preferred_element_type=jnp.float32)
        logits = logits + bg_ref[...]
        m = jnp.max(logits, axis=1, keepdims=True)
        p = jnp.exp(logits - m)
        probs_s[...] = p / jnp.sum(p, axis=1, keepdims=True)

    p = probs_s[...]
    lane = jax.lax.broadcasted_iota(jnp.int32, (TB, E_F), 1)
    m1 = jnp.max(p, axis=1, keepdims=True)
    a1 = jnp.min(jnp.where(p == m1, lane, E_F), axis=1, keepdims=True)
    p2 = jnp.where(lane == a1, -1.0, p)
    m2 = jnp.max(p2, axis=1, keepdims=True)
    a2 = jnp.min(jnp.where(p2 == m2, lane, E_F), axis=1, keepdims=True)
    gate = (jnp.where(a1 == e, m1, 0.0) + jnp.where(a2 == e, m2, 0.0)) / (m1 + m2)

    h = jnp.dot(x, W1_ref[0], preferred_element_type=jnp.float32) + b1_ref[0]
    h = jnp.maximum(h, 0.0)
    h = jnp.dot(h, W2_ref[0], preferred_element_type=jnp.float32) + b2_ref[0]
    h = jnp.maximum(h, 0.0)
    contrib = gate * h

    @pl.when(e == 0)
    def _():
        yacc[...] = contrib

    @pl.when(e > 0)
    def _():
        yacc[...] += contrib

    @pl.when(e == E_F - 1)
    def _():
        out_ref[...] = (
            jnp.dot(yacc[...], Wout_ref[...], preferred_element_type=jnp.float32)
            + bout_ref[...])


@functools.partial(jax.jit, static_argnames=("interpret",))
def _moe_fused(x, Wg, bg, W1, b1, W2, b2, Wout, bout,
               interpret=False):
    B = x.shape[0]
    T = B // TB
    grid = (T, E_F)
    out = pl.pallas_call(
        _fused_body,
        grid=grid,
        in_specs=[
            pl.BlockSpec((TB, D_FLAT_F), lambda t, e: (t, 0)),
            pl.BlockSpec((D_FLAT_F, E_F), lambda t, e: (0, 0)),
            pl.BlockSpec((1, E_F), lambda t, e: (0, 0)),
            pl.BlockSpec((1, D_FLAT_F, D_BLK), lambda t, e: (e, 0, 0)),
            pl.BlockSpec((1, 1, D_BLK), lambda t, e: (e, 0, 0)),
            pl.BlockSpec((1, D_BLK, D_BLK), lambda t, e: (e, 0, 0)),
            pl.BlockSpec((1, 1, D_BLK), lambda t, e: (e, 0, 0)),
            pl.BlockSpec((D_BLK, 1), lambda t, e: (0, 0)),
            pl.BlockSpec((1, 1), lambda t, e: (0, 0)),
        ],
        out_specs=pl.BlockSpec((TB, 1), lambda t, e: (t, 0)),
        out_shape=jax.ShapeDtypeStruct((B, 1), jnp.float32),
        scratch_shapes=[
            pltpu.VMEM((TB, D_BLK), jnp.float32),
            pltpu.VMEM((TB, E_F), jnp.float32),
        ],
        compiler_params=pltpu.CompilerParams(
            dimension_semantics=("parallel", "arbitrary")),
        interpret=interpret,
    )(x, Wg, bg.reshape(1, E_F), W1, b1.reshape(E_F, 1, D_BLK),
      W2, b2.reshape(E_F, 1, D_BLK), Wout, bout.reshape(1, 1))
    return out


def _build_x(x_num, x_cat):
    oh = jax.nn.one_hot(x_cat, CARD_F, dtype=jnp.float32)
    oh = oh.reshape(x_cat.shape[0], N_CAT_F * CARD_F)
    return jnp.concatenate([x_num, oh], axis=1)


def kernel(x_num, x_cat, Wg, bg, W1, b1, W2, b2, Wout, bout):
    x = _build_x(x_num, x_cat)
    out = _moe_fused(x, Wg, bg, W1, b1, W2, b2, Wout, bout)
    return out[:, None]
